# Initial kernel scaffold; baseline (speedup 1.0000x reference)
import jax
import jax.numpy as jnp
from jax.experimental import pallas as pl


def kernel(x, table, pos_encoding):
    raise NotImplementedError("write your pallas kernel here")


# speedup vs baseline: 4.9992x; 4.9992x over previous
import functools

import jax
import jax.numpy as jnp
from jax import lax
from jax.experimental import pallas as pl
from jax.experimental.pallas import tpu as pltpu
from jax.experimental.pallas import tpu_sc as plsc

_B, _L, _D, _V = 1024, 200, 128, 5
_NC, _NS = 2, 16
_NW = _NC * _NS
_TOK = _B * _L
_TPW = _TOK // _NW
_CHUNK = 128
_NCHUNK = _TPW // _CHUNK
_NBUF = 2


def _fuse_body(tab_ref, pe_ref, out_ref):
    out_ref[...] = tab_ref[...][:, None, :] + pe_ref[...][None, :, :]


def _build_combined(table, pe2d):
    return pl.pallas_call(
        _fuse_body,
        out_shape=jax.ShapeDtypeStruct((_V, _L, _D), jnp.float32),
    )(table, pe2d)


@functools.cache
def _make_sc_gather():
    mesh = plsc.VectorSubcoreMesh(core_axis_name="c", subcore_axis_name="s")
    return pl.kernel(
        _sc_gather_body,
        mesh=mesh,
        out_type=jax.ShapeDtypeStruct((_TOK, _D), jnp.float32),
        scratch_types=[
            pltpu.VMEM((_TPW,), jnp.int32),
            pltpu.VMEM((_TPW,), jnp.int32),
            pltpu.VMEM((_TPW,), jnp.int32),
            pltpu.VMEM((_NBUF, _CHUNK, _D), jnp.float32),
            pltpu.SemaphoreType.DMA,
            pltpu.SemaphoreType.DMA,
            pltpu.SemaphoreType.DMA,
            pltpu.SemaphoreType.DMA,
        ],
    )


def _sc_gather_body(comb_hbm, x_hbm, loff_hbm, out_hbm,
                    x_v, loff_v, idx_v, bufs, g0, g1, o0, o1):
    gsems = (g0, g1)
    osems = (o0, o1)
    wid = lax.axis_index("s") * _NC + lax.axis_index("c")
    base = wid * _TPW

    pltpu.sync_copy(x_hbm.at[pl.ds(base, _TPW)], x_v)
    pltpu.sync_copy(loff_hbm, loff_v)

    def idx_body(i, carry):
        s = pl.ds(i * 16, 16)
        idx_v[s] = x_v[s] * _L + loff_v[s]
        return carry

    lax.fori_loop(0, _TPW // 16, idx_body, 0)

    def fire_gather(c, b):
        pltpu.async_copy(
            comb_hbm.at[idx_v.at[pl.ds(c * _CHUNK, _CHUNK)]],
            bufs.at[b], gsems[b])

    def wait_gather(b):
        pltpu.make_async_copy(
            comb_hbm.at[idx_v.at[pl.ds(0, _CHUNK)]],
            bufs.at[b], gsems[b]).wait()

    def fire_out(c, b):
        pltpu.async_copy(
            bufs.at[b], out_hbm.at[pl.ds(base + c * _CHUNK, _CHUNK)],
            osems[b])

    def wait_out(b):
        pltpu.make_async_copy(
            bufs.at[b], out_hbm.at[pl.ds(base, _CHUNK)], osems[b]).wait()

    for b in range(_NBUF):
        fire_gather(b, b)

    def round_body(g, carry):
        for b in range(_NBUF):
            c = g * _NBUF + b
            wait_gather(b)
            fire_out(c, b)

            @pl.when(c + _NBUF < _NCHUNK)
            def _():
                wait_out(b)
                fire_gather(c + _NBUF, b)
        return carry

    lax.fori_loop(0, _NCHUNK // _NBUF, round_body, 0)
    for b in range(_NBUF):
        wait_out(b)


def kernel(x, table, pos_encoding):
    pe2d = pos_encoding[0, :_L, :]
    comb = _build_combined(table, pe2d).reshape(_V * _L, _D)
    x_flat = x.reshape(_TOK)
    loff = jnp.tile(jnp.arange(_L, dtype=jnp.int32), _TPW // _L)
    out_flat = _make_sc_gather()(comb, x_flat, loff)
    return out_flat.reshape(_B, _L, _D)


# speedup vs baseline: 10.7124x; 2.1428x over previous
import functools

import jax
import jax.numpy as jnp
from jax import lax
from jax.experimental import pallas as pl
from jax.experimental.pallas import tpu as pltpu
from jax.experimental.pallas import tpu_sc as plsc

_B, _L, _D, _V = 1024, 200, 128, 5
_NC, _NS = 2, 16
_NW = _NC * _NS
_TOK = _B * _L
_TPW = _TOK // _NW
_CHUNK = 128
_NCHUNK = _TPW // _CHUNK
_NBUF = 2


def _fuse_body(tab_ref, pe_ref, out_ref):
    out_ref[...] = tab_ref[...][:, None, :] + pe_ref[...][None, :, :]


def _build_combined(table, pe2d):
    return pl.pallas_call(
        _fuse_body,
        out_shape=jax.ShapeDtypeStruct((_V, _L, _D), jnp.float32),
    )(table, pe2d)


@functools.cache
def _make_sc_gather():
    mesh = plsc.VectorSubcoreMesh(core_axis_name="c", subcore_axis_name="s")
    return pl.kernel(
        _sc_gather_body,
        mesh=mesh,
        out_type=jax.ShapeDtypeStruct((_TOK, _D), jnp.float32),
        scratch_types=[
            pltpu.VMEM((_TPW,), jnp.int32),
            pltpu.VMEM((_TPW,), jnp.int32),
            pltpu.VMEM((_TPW,), jnp.int32),
            pltpu.VMEM((_NBUF, _CHUNK, _D), jnp.float32),
            pltpu.VMEM_SHARED((_V * _L, _D), jnp.float32),
            pltpu.SemaphoreType.DMA,
            pltpu.SemaphoreType.DMA,
            pltpu.SemaphoreType.DMA,
            pltpu.SemaphoreType.DMA,
        ],
    )


def _sc_gather_body(comb_hbm, x_hbm, loff_hbm, out_hbm,
                    x_v, loff_v, idx_v, bufs, comb_sp, g0, g1, o0, o1):
    gsems = (g0, g1)
    osems = (o0, o1)
    sid = lax.axis_index("s")
    wid = sid * _NC + lax.axis_index("c")
    base = wid * _TPW

    @pl.when(sid == 0)
    def _():
        pltpu.sync_copy(comb_hbm, comb_sp)

    pltpu.sync_copy(x_hbm.at[pl.ds(base, _TPW)], x_v)
    pltpu.sync_copy(loff_hbm, loff_v)

    def idx_body(i, carry):
        s = pl.ds(i * 16, 16)
        idx_v[s] = x_v[s] * _L + loff_v[s]
        return carry

    lax.fori_loop(0, _TPW // 16, idx_body, 0)

    plsc.subcore_barrier()

    def fire_gather(c, b):
        pltpu.async_copy(
            comb_sp.at[idx_v.at[pl.ds(c * _CHUNK, _CHUNK)]],
            bufs.at[b], gsems[b])

    def wait_gather(b):
        pltpu.make_async_copy(
            comb_sp.at[idx_v.at[pl.ds(0, _CHUNK)]],
            bufs.at[b], gsems[b]).wait()

    def fire_out(c, b):
        pltpu.async_copy(
            bufs.at[b], out_hbm.at[pl.ds(base + c * _CHUNK, _CHUNK)],
            osems[b])

    def wait_out(b):
        pltpu.make_async_copy(
            bufs.at[b], out_hbm.at[pl.ds(base, _CHUNK)], osems[b]).wait()

    for b in range(_NBUF):
        fire_gather(b, b)

    def round_body(g, carry):
        for b in range(_NBUF):
            c = g * _NBUF + b
            wait_gather(b)
            fire_out(c, b)

            @pl.when(c + _NBUF < _NCHUNK)
            def _():
                wait_out(b)
                fire_gather(c + _NBUF, b)
        return carry

    lax.fori_loop(0, _NCHUNK // _NBUF, round_body, 0)
    for b in range(_NBUF):
        wait_out(b)


def kernel(x, table, pos_encoding):
    pe2d = pos_encoding[0, :_L, :]
    comb = _build_combined(table, pe2d).reshape(_V * _L, _D)
    x_flat = x.reshape(_TOK)
    loff = jnp.tile(jnp.arange(_L, dtype=jnp.int32), _TPW // _L)
    out_flat = _make_sc_gather()(comb, x_flat, loff)
    return out_flat.reshape(_B, _L, _D)
